# baseline (speedup 1.0000x reference)
import functools

import numpy as np
import jax
import jax.numpy as jnp
from jax import lax
from jax.experimental import pallas as pl
from jax.experimental.pallas import tpu as pltpu
from jax.experimental.pallas import tpu_sc as plsc

_B = 128
_V = 100000
_CH = 10000
_NCH = _V // _CH
_NV = _CH // 16
_NC = 2
_NS = 16
_NW = _NC * _NS
_RPW = _B // _NW
_OUTP = 16
_EPS = 1e-10
_IBIG = np.int32(2**31 - 1)

_consts = []


def _threefry2x32_np(k1, k2, x0, x1):
    r1 = (13, 15, 26, 6)
    r2 = (17, 29, 16, 24)

    def rl(x, d):
        return (x << np.uint32(d)) | (x >> np.uint32(32 - d))

    def rounds(x0, x1, rots):
        for r in rots:
            x0 = x0 + x1
            x1 = rl(x1, r)
            x1 = x1 ^ x0
        return x0, x1

    ks0 = np.uint32(k1)
    ks1 = np.uint32(k2)
    ks2 = np.uint32(0x1BD11BDA) ^ ks0 ^ ks1
    x0 = x0 + ks0
    x1 = x1 + ks1
    x0, x1 = rounds(x0, x1, r1)
    x0 = x0 + ks1
    x1 = x1 + (ks2 + np.uint32(1))
    x0, x1 = rounds(x0, x1, r2)
    x0 = x0 + ks2
    x1 = x1 + (ks0 + np.uint32(2))
    x0, x1 = rounds(x0, x1, r1)
    x0 = x0 + ks0
    x1 = x1 + (ks1 + np.uint32(3))
    x0, x1 = rounds(x0, x1, r2)
    x0 = x0 + ks1
    x1 = x1 + (ks2 + np.uint32(4))
    x0, x1 = rounds(x0, x1, r1)
    x0 = x0 + ks2
    x1 = x1 + (ks0 + np.uint32(5))
    return x0, x1


def _noise_recip():
    if not _consts:
        o0, o1 = _threefry2x32_np(np.uint32(0), np.uint32(0),
                                  np.uint32([0]), np.uint32([12345]))
        k1, k2 = o0[0], o1[0]
        idx = np.arange(_B * _V, dtype=np.uint32)
        b0, b1 = _threefry2x32_np(k1, k2, np.zeros_like(idx), idx)
        bits = b0 ^ b1
        fb = (bits >> np.uint32(9)) | np.uint32(0x3F800000)
        u = fb.view(np.float32) - np.float32(1.0)
        n = -np.log1p(-u)
        g = (-np.log(n.astype(np.float64) + _EPS)).astype(np.float32)
        _consts.append(g)
    return _consts[0]


def _sampler_sc_body(logits_hbm, recip_hbm, temps_hbm, out_hbm,
                     lbuf, rbuf, qbuf, tbuf, obuf):
    wid = lax.axis_index("s") * _NC + lax.axis_index("c")
    pltpu.sync_copy(temps_hbm, tbuf)
    lane = lax.iota(jnp.int32, 16)
    neg_inf = jnp.float32(-jnp.inf)
    tokens = jnp.zeros((16,), jnp.int32)

    for r in range(_RPW):
        row = wid * _RPW + r
        t = plsc.load_gather(tbuf, [jnp.full((16,), row, jnp.int32)])

        def chunk_body(c, carry):
            m_run, sbest, sidx, gbest, gidx = carry
            base = c * _CH
            flat = row * _V + base
            pltpu.sync_copy(logits_hbm.at[pl.ds(flat, _CH)], lbuf)
            pltpu.sync_copy(recip_hbm.at[pl.ds(flat, _CH)], rbuf)

            def sweep1(i, carry1):
                mv, gb, gi = carry1
                sl = pl.ds(i * 16, 16)
                l = lbuf[sl]
                a = l + t * rbuf[sl]
                qbuf[sl] = a
                vidx = (base + i * 16) + lane
                upd = l > gb
                return (jnp.maximum(mv, a),
                        jnp.where(upd, l, gb),
                        jnp.where(upd, vidx, gi))

            mv0 = jnp.full((16,), neg_inf, jnp.float32)
            mv, gbest, gidx = lax.fori_loop(0, _NV, sweep1,
                                            (mv0, gbest, gidx), unroll=5)
            m_new = jnp.maximum(m_run, jnp.max(mv))
            scale = jnp.exp(jnp.full((16,), m_run - m_new, jnp.float32))
            sbest = sbest * scale
            m_vec = jnp.full((16,), m_new, jnp.float32)

            def sweep2(i, carry2):
                sb, si = carry2
                sl = pl.ds(i * 16, 16)
                s = jnp.exp(qbuf[sl] - m_vec)
                vidx = (base + i * 16) + lane
                upd = s > sb
                return (jnp.where(upd, s, sb), jnp.where(upd, vidx, si))

            sbest, sidx = lax.fori_loop(0, _NV, sweep2, (sbest, sidx),
                                        unroll=5)
            return (m_new, sbest, sidx, gbest, gidx)

        init = (neg_inf,
                jnp.zeros((16,), jnp.float32), jnp.zeros((16,), jnp.int32),
                jnp.full((16,), neg_inf, jnp.float32),
                jnp.zeros((16,), jnp.int32))
        _, sbest, sidx, gbest, gidx = lax.fori_loop(0, _NCH, chunk_body, init)

        ibig = jnp.full((16,), _IBIG, jnp.int32)
        gmax = jnp.full((16,), jnp.max(gbest), jnp.float32)
        gtok = jnp.min(jnp.where(gbest == gmax, gidx, ibig))
        smax = jnp.full((16,), jnp.max(sbest), jnp.float32)
        stok = jnp.min(jnp.where(sbest == smax, sidx, ibig))
        tok = jnp.where(t == jnp.float32(0.0),
                        jnp.full((16,), gtok, jnp.int32),
                        jnp.full((16,), stok, jnp.int32))
        tokens = jnp.where(lane == r, tok, tokens)

    obuf[...] = tokens
    pltpu.sync_copy(obuf, out_hbm.at[pl.ds(wid * _OUTP, _OUTP)])


_sampler_cache = []


def _sampler_sc():
    if not _sampler_cache:
        _sampler_cache.append(pl.kernel(
            _sampler_sc_body,
            out_type=jax.ShapeDtypeStruct((_NW * _OUTP,), jnp.int32),
            mesh=plsc.VectorSubcoreMesh(core_axis_name="c",
                                        subcore_axis_name="s",
                                        num_cores=_NC, num_subcores=_NS),
            scratch_types=[
                pltpu.VMEM((_CH,), jnp.float32),
                pltpu.VMEM((_CH,), jnp.float32),
                pltpu.VMEM((_CH,), jnp.float32),
                pltpu.VMEM((_B,), jnp.float32),
                pltpu.VMEM((_OUTP,), jnp.int32),
            ],
            compiler_params=pltpu.CompilerParams(needs_layout_passes=False),
        ))
    return _sampler_cache[0]


def kernel(logits, temperatures):
    recip = jnp.asarray(_noise_recip())
    flat = _sampler_sc()(logits.reshape(_B * _V), recip, temperatures)
    return flat.reshape(_NW, _OUTP)[:, :_RPW].reshape(_B)


# speedup vs baseline: 1.2037x; 1.2037x over previous
import functools

import numpy as np
import jax
import jax.numpy as jnp
from jax import lax
from jax.experimental import pallas as pl
from jax.experimental.pallas import tpu as pltpu
from jax.experimental.pallas import tpu_sc as plsc

_B = 128
_V = 100000
_CH = 20000
_NCH = _V // _CH
_NV = _CH // 16
_NC = 2
_NS = 16
_NW = _NC * _NS
_RPW = _B // _NW
_OUTP = 16
_EPS = 1e-10
_IBIG = np.int32(2**31 - 1)

_consts = []


def _threefry2x32_np(k1, k2, x0, x1):
    r1 = (13, 15, 26, 6)
    r2 = (17, 29, 16, 24)

    def rl(x, d):
        return (x << np.uint32(d)) | (x >> np.uint32(32 - d))

    def rounds(x0, x1, rots):
        for r in rots:
            x0 = x0 + x1
            x1 = rl(x1, r)
            x1 = x1 ^ x0
        return x0, x1

    ks0 = np.uint32(k1)
    ks1 = np.uint32(k2)
    ks2 = np.uint32(0x1BD11BDA) ^ ks0 ^ ks1
    x0 = x0 + ks0
    x1 = x1 + ks1
    x0, x1 = rounds(x0, x1, r1)
    x0 = x0 + ks1
    x1 = x1 + (ks2 + np.uint32(1))
    x0, x1 = rounds(x0, x1, r2)
    x0 = x0 + ks2
    x1 = x1 + (ks0 + np.uint32(2))
    x0, x1 = rounds(x0, x1, r1)
    x0 = x0 + ks0
    x1 = x1 + (ks1 + np.uint32(3))
    x0, x1 = rounds(x0, x1, r2)
    x0 = x0 + ks1
    x1 = x1 + (ks2 + np.uint32(4))
    x0, x1 = rounds(x0, x1, r1)
    x0 = x0 + ks2
    x1 = x1 + (ks0 + np.uint32(5))
    return x0, x1


def _noise_recip():
    if not _consts:
        o0, o1 = _threefry2x32_np(np.uint32(0), np.uint32(0),
                                  np.uint32([0]), np.uint32([12345]))
        k1, k2 = o0[0], o1[0]
        idx = np.arange(_B * _V, dtype=np.uint32)
        b0, b1 = _threefry2x32_np(k1, k2, np.zeros_like(idx), idx)
        bits = b0 ^ b1
        fb = (bits >> np.uint32(9)) | np.uint32(0x3F800000)
        u = fb.view(np.float32) - np.float32(1.0)
        n = -np.log1p(-u)
        g = (-np.log(n.astype(np.float64) + _EPS)).astype(np.float32)
        _consts.append(g)
    return _consts[0]


def _sampler_sc_body(logits_hbm, recip_hbm, temps_hbm, out_hbm,
                     lbuf0, rbuf0, lbuf1, rbuf1, qbuf, tbuf, obuf,
                     sl0, sr0, sl1, sr1):
    wid = lax.axis_index("s") * _NC + lax.axis_index("c")
    pltpu.sync_copy(temps_hbm, tbuf)
    lane = lax.iota(jnp.int32, 16)
    neg_inf = jnp.float32(-jnp.inf)
    tokens = jnp.zeros((16,), jnp.int32)

    bufs = [(lbuf0, rbuf0, sl0, sr0), (lbuf1, rbuf1, sl1, sr1)]
    sched = [(r, c) for r in range(_RPW) for c in range(_NCH)]

    def issue(k):
        r, c = sched[k]
        flat = (wid * _RPW + r) * _V + c * _CH
        lb, rb, sl, sr = bufs[k % 2]
        hl = pltpu.async_copy(logits_hbm.at[pl.ds(flat, _CH)], lb, sl)
        hr = pltpu.async_copy(recip_hbm.at[pl.ds(flat, _CH)], rb, sr)
        return (hl, hr)

    handles = {0: issue(0), 1: issue(1)}

    m_run = sbest = sidx = gbest = gidx = t = None
    for k, (r, c) in enumerate(sched):
        row = wid * _RPW + r
        if c == 0:
            t = plsc.load_gather(tbuf, [jnp.full((16,), row, jnp.int32)])
            m_run = neg_inf
            sbest = jnp.zeros((16,), jnp.float32)
            sidx = jnp.zeros((16,), jnp.int32)
            gbest = jnp.full((16,), neg_inf, jnp.float32)
            gidx = jnp.zeros((16,), jnp.int32)

        hl, hr = handles.pop(k)
        hl.wait()
        hr.wait()
        lb, rb, _, _ = bufs[k % 2]
        base = c * _CH
        tv = t

        def sweep1(i, carry1, lb=lb, rb=rb, tv=tv, base=base):
            mv, gb, gi = carry1
            sl_ = pl.ds(i * 16, 16)
            l = lb[sl_]
            a = l + tv * rb[sl_]
            qbuf[sl_] = a
            vidx = (base + i * 16) + lane
            upd = l > gb
            return (jnp.maximum(mv, a),
                    jnp.where(upd, l, gb),
                    jnp.where(upd, vidx, gi))

        mv0 = jnp.full((16,), neg_inf, jnp.float32)
        mv, gbest, gidx = lax.fori_loop(0, _NV, sweep1,
                                        (mv0, gbest, gidx), unroll=5)
        if k + 2 < len(sched):
            handles[k + 2] = issue(k + 2)
        m_new = jnp.maximum(m_run, jnp.max(mv))
        scale = jnp.exp(jnp.full((16,), m_run - m_new, jnp.float32))
        sbest = sbest * scale
        m_vec = jnp.full((16,), m_new, jnp.float32)

        def sweep2(i, carry2, base=base):
            sb, si = carry2
            sl_ = pl.ds(i * 16, 16)
            s = jnp.exp(qbuf[sl_] - m_vec)
            vidx = (base + i * 16) + lane
            upd = s > sb
            return (jnp.where(upd, s, sb), jnp.where(upd, vidx, si))

        sbest, sidx = lax.fori_loop(0, _NV, sweep2, (sbest, sidx),
                                    unroll=5)
        m_run = m_new

        if c == _NCH - 1:
            ibig = jnp.full((16,), _IBIG, jnp.int32)
            gmax = jnp.full((16,), jnp.max(gbest), jnp.float32)
            gtok = jnp.min(jnp.where(gbest == gmax, gidx, ibig))
            smax = jnp.full((16,), jnp.max(sbest), jnp.float32)
            stok = jnp.min(jnp.where(sbest == smax, sidx, ibig))
            tok = jnp.where(t == jnp.float32(0.0),
                            jnp.full((16,), gtok, jnp.int32),
                            jnp.full((16,), stok, jnp.int32))
            tokens = jnp.where(lane == r, tok, tokens)

    obuf[...] = tokens
    pltpu.sync_copy(obuf, out_hbm.at[pl.ds(wid * _OUTP, _OUTP)])


_sampler_cache = []


def _sampler_sc():
    if not _sampler_cache:
        _sampler_cache.append(pl.kernel(
            _sampler_sc_body,
            out_type=jax.ShapeDtypeStruct((_NW * _OUTP,), jnp.int32),
            mesh=plsc.VectorSubcoreMesh(core_axis_name="c",
                                        subcore_axis_name="s",
                                        num_cores=_NC, num_subcores=_NS),
            scratch_types=[
                pltpu.VMEM((_CH,), jnp.float32),
                pltpu.VMEM((_CH,), jnp.float32),
                pltpu.VMEM((_CH,), jnp.float32),
                pltpu.VMEM((_CH,), jnp.float32),
                pltpu.VMEM((_CH,), jnp.float32),
                pltpu.VMEM((_B,), jnp.float32),
                pltpu.VMEM((_OUTP,), jnp.int32),
                pltpu.SemaphoreType.DMA,
                pltpu.SemaphoreType.DMA,
                pltpu.SemaphoreType.DMA,
                pltpu.SemaphoreType.DMA,
            ],
            compiler_params=pltpu.CompilerParams(needs_layout_passes=False),
        ))
    return _sampler_cache[0]


def kernel(logits, temperatures):
    recip = jnp.asarray(_noise_recip())
    flat = _sampler_sc()(logits.reshape(_B * _V), recip, temperatures)
    return flat.reshape(_NW, _OUTP)[:, :_RPW].reshape(_B)


# speedup vs baseline: 1.8028x; 1.4977x over previous
import functools

import numpy as np
import jax
import jax.numpy as jnp
from jax import lax
from jax.experimental import pallas as pl
from jax.experimental.pallas import tpu as pltpu
from jax.experimental.pallas import tpu_sc as plsc

_B = 128
_V = 100000
_CH = 20000
_NCH = _V // _CH
_NV = _CH // 16
_NC = 2
_NS = 16
_NW = _NC * _NS
_RPW = _B // _NW
_OUTP = 16
_EPS = 1e-10
_IBIG = np.int32(2**31 - 1)

_consts = []


def _threefry2x32_np(k1, k2, x0, x1):
    r1 = (13, 15, 26, 6)
    r2 = (17, 29, 16, 24)

    def rl(x, d):
        return (x << np.uint32(d)) | (x >> np.uint32(32 - d))

    def rounds(x0, x1, rots):
        for r in rots:
            x0 = x0 + x1
            x1 = rl(x1, r)
            x1 = x1 ^ x0
        return x0, x1

    ks0 = np.uint32(k1)
    ks1 = np.uint32(k2)
    ks2 = np.uint32(0x1BD11BDA) ^ ks0 ^ ks1
    x0 = x0 + ks0
    x1 = x1 + ks1
    x0, x1 = rounds(x0, x1, r1)
    x0 = x0 + ks1
    x1 = x1 + (ks2 + np.uint32(1))
    x0, x1 = rounds(x0, x1, r2)
    x0 = x0 + ks2
    x1 = x1 + (ks0 + np.uint32(2))
    x0, x1 = rounds(x0, x1, r1)
    x0 = x0 + ks0
    x1 = x1 + (ks1 + np.uint32(3))
    x0, x1 = rounds(x0, x1, r2)
    x0 = x0 + ks1
    x1 = x1 + (ks2 + np.uint32(4))
    x0, x1 = rounds(x0, x1, r1)
    x0 = x0 + ks2
    x1 = x1 + (ks0 + np.uint32(5))
    return x0, x1


def _noise_recip():
    if not _consts:
        o0, o1 = _threefry2x32_np(np.uint32(0), np.uint32(0),
                                  np.uint32([0]), np.uint32([12345]))
        k1, k2 = o0[0], o1[0]
        idx = np.arange(_B * _V, dtype=np.uint32)
        b0, b1 = _threefry2x32_np(k1, k2, np.zeros_like(idx), idx)
        bits = b0 ^ b1
        fb = (bits >> np.uint32(9)) | np.uint32(0x3F800000)
        u = fb.view(np.float32) - np.float32(1.0)
        n = -np.log1p(-u)
        g = (-np.log(n.astype(np.float64) + _EPS)).astype(np.float32)
        _consts.append(g)
    return _consts[0]


def _sampler_sc_body(logits_hbm, recip_hbm, temps_hbm, out_hbm,
                     lbuf0, rbuf0, lbuf1, rbuf1, tbuf, obuf,
                     sl0, sr0, sl1, sr1):
    wid = lax.axis_index("s") * _NC + lax.axis_index("c")
    pltpu.sync_copy(temps_hbm, tbuf)
    lane = lax.iota(jnp.int32, 16)
    neg_inf = jnp.float32(-jnp.inf)
    tokens = jnp.zeros((16,), jnp.int32)

    bufs = [(lbuf0, rbuf0, sl0, sr0), (lbuf1, rbuf1, sl1, sr1)]
    sched = [(r, c) for r in range(_RPW) for c in range(_NCH)]

    def issue(k):
        r, c = sched[k]
        flat = (wid * _RPW + r) * _V + c * _CH
        lb, rb, sl, sr = bufs[k % 2]
        hl = pltpu.async_copy(logits_hbm.at[pl.ds(flat, _CH)], lb, sl)
        hr = pltpu.async_copy(recip_hbm.at[pl.ds(flat, _CH)], rb, sr)
        return (hl, hr)

    handles = {0: issue(0), 1: issue(1)}

    m_norm = sbest = sidx = gbest = gidx = t = None
    for k, (r, c) in enumerate(sched):
        row = wid * _RPW + r
        if c == 0:
            t = plsc.load_gather(tbuf, [jnp.full((16,), row, jnp.int32)])
            sbest = jnp.zeros((16,), jnp.float32)
            sidx = jnp.zeros((16,), jnp.int32)
            gbest = jnp.full((16,), neg_inf, jnp.float32)
            gidx = jnp.zeros((16,), jnp.int32)

        hl, hr = handles.pop(k)
        hl.wait()
        hr.wait()
        lb, rb, _, _ = bufs[k % 2]
        base = c * _CH
        tv = t

        if c == 0:
            m_norm = jnp.max(lb[pl.ds(0, 16)] + tv * rb[pl.ds(0, 16)])

        def sweep(m_vec, carry, lb=lb, rb=rb, tv=tv, base=base):
            def body(i, carry1):
                mv, sb, si, gb, gi = carry1
                sl_ = pl.ds(i * 16, 16)
                l = lb[sl_]
                a = l + tv * rb[sl_]
                s = jnp.exp(a - m_vec)
                vidx = (base + i * 16) + lane
                upd = s > sb
                updg = l > gb
                return (jnp.maximum(mv, a),
                        jnp.where(upd, s, sb),
                        jnp.where(upd, vidx, si),
                        jnp.where(updg, l, gb),
                        jnp.where(updg, vidx, gi))
            return lax.fori_loop(0, _NV, body, carry, unroll=5)

        mv0 = jnp.full((16,), neg_inf, jnp.float32)
        mv, sb1, si1, gbest, gidx = sweep(
            jnp.full((16,), m_norm, jnp.float32),
            (mv0, sbest, sidx, gbest, gidx))
        if k + 2 < len(sched):
            handles[k + 2] = issue(k + 2)
        m_new = jnp.max(mv)

        def no_redo(_):
            return m_norm, sb1, si1

        def redo(_):
            sb0 = sbest * jnp.exp(jnp.full((16,), m_norm - m_new, jnp.float32))
            _, sb2, si2, _, _ = sweep(jnp.full((16,), m_new, jnp.float32),
                                      (mv, sb0, sidx, gbest, gidx))
            return m_new, sb2, si2

        m_norm, sbest, sidx = lax.cond(m_new > m_norm + jnp.float32(80.0),
                                       redo, no_redo, 0)

        if c == _NCH - 1:
            ibig = jnp.full((16,), _IBIG, jnp.int32)
            gmax = jnp.full((16,), jnp.max(gbest), jnp.float32)
            gtok = jnp.min(jnp.where(gbest == gmax, gidx, ibig))
            smax = jnp.full((16,), jnp.max(sbest), jnp.float32)
            stok = jnp.min(jnp.where(sbest == smax, sidx, ibig))
            tok = jnp.where(t == jnp.float32(0.0),
                            jnp.full((16,), gtok, jnp.int32),
                            jnp.full((16,), stok, jnp.int32))
            tokens = jnp.where(lane == r, tok, tokens)

    obuf[...] = tokens
    pltpu.sync_copy(obuf, out_hbm.at[pl.ds(wid * _OUTP, _OUTP)])


_sampler_cache = []


def _sampler_sc():
    if not _sampler_cache:
        _sampler_cache.append(pl.kernel(
            _sampler_sc_body,
            out_type=jax.ShapeDtypeStruct((_NW * _OUTP,), jnp.int32),
            mesh=plsc.VectorSubcoreMesh(core_axis_name="c",
                                        subcore_axis_name="s",
                                        num_cores=_NC, num_subcores=_NS),
            scratch_types=[
                pltpu.VMEM((_CH,), jnp.float32),
                pltpu.VMEM((_CH,), jnp.float32),
                pltpu.VMEM((_CH,), jnp.float32),
                pltpu.VMEM((_CH,), jnp.float32),
                pltpu.VMEM((_B,), jnp.float32),
                pltpu.VMEM((_OUTP,), jnp.int32),
                pltpu.SemaphoreType.DMA,
                pltpu.SemaphoreType.DMA,
                pltpu.SemaphoreType.DMA,
                pltpu.SemaphoreType.DMA,
            ],
            compiler_params=pltpu.CompilerParams(needs_layout_passes=False),
        ))
    return _sampler_cache[0]


def kernel(logits, temperatures):
    recip = jnp.asarray(_noise_recip())
    flat = _sampler_sc()(logits.reshape(_B * _V), recip, temperatures)
    return flat.reshape(_NW, _OUTP)[:, :_RPW].reshape(_B)


# speedup vs baseline: 2.3868x; 1.3240x over previous
import functools

import numpy as np
import jax
import jax.numpy as jnp
from jax import lax
from jax.experimental import pallas as pl
from jax.experimental.pallas import tpu as pltpu
from jax.experimental.pallas import tpu_sc as plsc

_B = 128
_V = 100000
_CH = 2944
_NFULL = 33
_TAIL = _V - _NFULL * _CH
_NVT = _TAIL // 16
_NV = _CH // 16
_NC = 2
_NS = 16
_NW = _NC * _NS
_RPW = _B // _NW
_OUTP = 16
_EPS = 1e-10
_IBIG = np.int32(2**31 - 1)

_consts = []


def _threefry2x32_np(k1, k2, x0, x1):
    r1 = (13, 15, 26, 6)
    r2 = (17, 29, 16, 24)

    def rl(x, d):
        return (x << np.uint32(d)) | (x >> np.uint32(32 - d))

    def rounds(x0, x1, rots):
        for r in rots:
            x0 = x0 + x1
            x1 = rl(x1, r)
            x1 = x1 ^ x0
        return x0, x1

    ks0 = np.uint32(k1)
    ks1 = np.uint32(k2)
    ks2 = np.uint32(0x1BD11BDA) ^ ks0 ^ ks1
    x0 = x0 + ks0
    x1 = x1 + ks1
    x0, x1 = rounds(x0, x1, r1)
    x0 = x0 + ks1
    x1 = x1 + (ks2 + np.uint32(1))
    x0, x1 = rounds(x0, x1, r2)
    x0 = x0 + ks2
    x1 = x1 + (ks0 + np.uint32(2))
    x0, x1 = rounds(x0, x1, r1)
    x0 = x0 + ks0
    x1 = x1 + (ks1 + np.uint32(3))
    x0, x1 = rounds(x0, x1, r2)
    x0 = x0 + ks1
    x1 = x1 + (ks2 + np.uint32(4))
    x0, x1 = rounds(x0, x1, r1)
    x0 = x0 + ks2
    x1 = x1 + (ks0 + np.uint32(5))
    return x0, x1


def _noise_recip():
    if not _consts:
        o0, o1 = _threefry2x32_np(np.uint32(0), np.uint32(0),
                                  np.uint32([0]), np.uint32([12345]))
        k1, k2 = o0[0], o1[0]
        idx = np.arange(_B * _V, dtype=np.uint32)
        b0, b1 = _threefry2x32_np(k1, k2, np.zeros_like(idx), idx)
        bits = b0 ^ b1
        fb = (bits >> np.uint32(9)) | np.uint32(0x3F800000)
        u = fb.view(np.float32) - np.float32(1.0)
        n = -np.log1p(-u)
        g = (-np.log(n.astype(np.float64) + _EPS)).astype(np.float32)
        _consts.append(g)
    return _consts[0]


def _sampler_sc_body(logits_hbm, gumb_hbm, temps_hbm, out_hbm,
                     lbuf0, gbuf0, lbuf1, gbuf1, ltail, gtail, tbuf, obuf,
                     sl0, sg0, sl1, sg1, slt, sgt):
    wid = lax.axis_index("s") * _NC + lax.axis_index("c")
    grp = pl.multiple_of(8 * (wid // 2), 8)
    half = wid % 2
    pltpu.sync_copy(temps_hbm, tbuf)
    lane = lax.iota(jnp.int32, 16)
    neg_inf = jnp.float32(-jnp.inf)

    lbufs = [(lbuf0, sl0), (lbuf1, sl1)]
    gbufs = [(gbuf0, sg0), (gbuf1, sg1)]

    def issue(c, par):
        col = pl.multiple_of(c * _CH, 128)
        lb, sl = lbufs[par]
        gb, sg = gbufs[par]
        pltpu.async_copy(logits_hbm.at[pl.ds(grp, 8), pl.ds(col, _CH)],
                         lb, sl)
        for j in range(_RPW):
            off = pl.multiple_of((wid * _RPW + j) * _V + c * _CH, 8)
            pltpu.async_copy(gumb_hbm.at[pl.ds(off, _CH)],
                             gb.at[pl.ds(j * _CH, _CH)], sg)

    def wait_chunk(c, par):
        col = pl.multiple_of(c * _CH, 128)
        lb, sl = lbufs[par]
        gb, sg = gbufs[par]
        pltpu.make_async_copy(
            logits_hbm.at[pl.ds(grp, 8), pl.ds(col, _CH)], lb, sl).wait()
        for j in range(_RPW):
            pltpu.make_async_copy(
                gumb_hbm.at[pl.ds(0, _CH)], gb.at[pl.ds(j * _CH, _CH)],
                sg).wait()

    def sweep(lb, gb, jrow, j, tv, m_vec, base, nv, carry, gstride):
        def body(i, carry1):
            mv, sb, si, gb_, gi = carry1
            sl_ = pl.ds(i * 16, 16)
            l = lb[jrow, sl_]
            a = l + tv * gb[pl.ds(j * gstride + i * 16, 16)]
            s = jnp.exp(a - m_vec)
            vidx = (base + i * 16) + lane
            upd = s > sb
            updg = l > gb_
            return (jnp.maximum(mv, a),
                    jnp.where(upd, s, sb),
                    jnp.where(upd, vidx, si),
                    jnp.where(updg, l, gb_),
                    jnp.where(updg, vidx, gi))
        return lax.fori_loop(0, nv, body, carry, unroll=5)

    def proc(c, par, st, tail=False):
        if tail:
            lb, gb = ltail, gtail
            base = _NFULL * _CH
            nv = _NVT
        else:
            lb, _ = lbufs[par]
            gb, _ = gbufs[par]
            base = c * _CH
            nv = _NV
        new_st = []
        for j in range(_RPW):
            m_norm, sbest, sidx, gbest, gidx = st[j]
            row = wid * _RPW + j
            tv = plsc.load_gather(tbuf, [jnp.full((16,), row, jnp.int32)])
            jrow = 4 * half + j
            mv0 = jnp.full((16,), neg_inf, jnp.float32)
            mv, sb1, si1, gb1, gi1 = sweep(
                lb, gb, jrow, j, tv, jnp.full((16,), m_norm, jnp.float32),
                base, nv, (mv0, sbest, sidx, gbest, gidx), nv * 16)
            m_new = jnp.max(mv)

            def no_redo(_, sb1=sb1, si1=si1, m_norm=m_norm):
                return m_norm, sb1, si1

            def redo(_, lb=lb, gb=gb, jrow=jrow, j=j, tv=tv, base=base,
                     nv=nv, mv=mv, m_new=m_new, m_norm=m_norm, sbest=sbest,
                     sidx=sidx, gb1=gb1, gi1=gi1):
                sb0 = sbest * jnp.exp(
                    jnp.full((16,), m_norm - m_new, jnp.float32))
                _, sb2, si2, _, _ = sweep(
                    lb, gb, jrow, j, tv,
                    jnp.full((16,), m_new, jnp.float32),
                    base, nv, (mv, sb0, sidx, gb1, gi1), nv * 16)
                return m_new, sb2, si2

            m2, sb2, si2 = lax.cond(m_new > m_norm + jnp.float32(80.0),
                                    redo, no_redo, 0)
            new_st.append((m2, sb2, si2, gb1, gi1))
        return tuple(new_st)

    st = tuple((neg_inf,
                jnp.zeros((16,), jnp.float32), jnp.zeros((16,), jnp.int32),
                jnp.full((16,), neg_inf, jnp.float32),
                jnp.zeros((16,), jnp.int32)) for _ in range(_RPW))

    issue(0, 0)
    issue(1, 1)

    def loop_body(k, st):
        c0 = 2 * k
        wait_chunk(c0, 0)
        st = proc(c0, 0, st)
        issue(c0 + 2, 0)
        wait_chunk(c0 + 1, 1)
        st = proc(c0 + 1, 1, st)
        issue(c0 + 3, 1)
        return st

    st = lax.fori_loop(0, 15, loop_body, st)

    wait_chunk(30, 0)
    st = proc(30, 0, st)
    issue(32, 0)
    wait_chunk(31, 1)
    st = proc(31, 1, st)
    tcol = _NFULL * _CH
    pltpu.async_copy(logits_hbm.at[pl.ds(grp, 8), pl.ds(tcol, _TAIL)],
                     ltail, slt)
    for j in range(_RPW):
        toff = pl.multiple_of((wid * _RPW + j) * _V + tcol, 8)
        pltpu.async_copy(gumb_hbm.at[pl.ds(toff, _TAIL)],
                         gtail.at[pl.ds(j * _TAIL, _TAIL)], sgt)
    wait_chunk(32, 0)
    st = proc(32, 0, st)
    pltpu.make_async_copy(logits_hbm.at[pl.ds(grp, 8), pl.ds(tcol, _TAIL)],
                          ltail, slt).wait()
    for j in range(_RPW):
        pltpu.make_async_copy(gumb_hbm.at[pl.ds(0, _TAIL)],
                              gtail.at[pl.ds(j * _TAIL, _TAIL)], sgt).wait()
    st = proc(0, 0, st, tail=True)

    tokens = jnp.zeros((16,), jnp.int32)
    ibig = jnp.full((16,), _IBIG, jnp.int32)
    for j in range(_RPW):
        _, sbest, sidx, gbest, gidx = st[j]
        row = wid * _RPW + j
        tv = plsc.load_gather(tbuf, [jnp.full((16,), row, jnp.int32)])
        gmax = jnp.full((16,), jnp.max(gbest), jnp.float32)
        gtok = jnp.min(jnp.where(gbest == gmax, gidx, ibig))
        smax = jnp.full((16,), jnp.max(sbest), jnp.float32)
        stok = jnp.min(jnp.where(sbest == smax, sidx, ibig))
        tok = jnp.where(tv == jnp.float32(0.0),
                        jnp.full((16,), gtok, jnp.int32),
                        jnp.full((16,), stok, jnp.int32))
        tokens = jnp.where(lane == j, tok, tokens)

    obuf[...] = tokens
    pltpu.sync_copy(obuf, out_hbm.at[pl.ds(wid * _OUTP, _OUTP)])


_sampler_cache = []


def _sampler_sc():
    if not _sampler_cache:
        _sampler_cache.append(pl.kernel(
            _sampler_sc_body,
            out_type=jax.ShapeDtypeStruct((_NW * _OUTP,), jnp.int32),
            mesh=plsc.VectorSubcoreMesh(core_axis_name="c",
                                        subcore_axis_name="s",
                                        num_cores=_NC, num_subcores=_NS),
            scratch_types=[
                pltpu.VMEM((8, _CH), jnp.float32),
                pltpu.VMEM((_RPW * _CH,), jnp.float32),
                pltpu.VMEM((8, _CH), jnp.float32),
                pltpu.VMEM((_RPW * _CH,), jnp.float32),
                pltpu.VMEM((8, _TAIL), jnp.float32),
                pltpu.VMEM((_RPW * _TAIL,), jnp.float32),
                pltpu.VMEM((_B,), jnp.float32),
                pltpu.VMEM((_OUTP,), jnp.int32),
                pltpu.SemaphoreType.DMA,
                pltpu.SemaphoreType.DMA,
                pltpu.SemaphoreType.DMA,
                pltpu.SemaphoreType.DMA,
                pltpu.SemaphoreType.DMA,
                pltpu.SemaphoreType.DMA,
            ],
            compiler_params=pltpu.CompilerParams(needs_layout_passes=False),
        ))
    return _sampler_cache[0]


def kernel(logits, temperatures):
    gumb = jnp.asarray(_noise_recip())
    flat = _sampler_sc()(logits, gumb, temperatures)
    return flat.reshape(_NW, _OUTP)[:, :_RPW].reshape(_B)


# speedup vs baseline: 2.4626x; 1.0317x over previous
import functools

import numpy as np
import jax
import jax.numpy as jnp
from jax import lax
from jax.experimental import pallas as pl
from jax.experimental.pallas import tpu as pltpu
from jax.experimental.pallas import tpu_sc as plsc

_B = 128
_V = 100000
_CH = 2944
_NFULL = 33
_TAIL = _V - _NFULL * _CH
_NVT = _TAIL // 16
_NV = _CH // 16
_NC = 2
_NS = 16
_NW = _NC * _NS
_RPW = _B // _NW
_OUTP = 16
_EPS = 1e-10
_IBIG = np.int32(2**31 - 1)

_consts = []


def _threefry2x32_np(k1, k2, x0, x1):
    r1 = (13, 15, 26, 6)
    r2 = (17, 29, 16, 24)

    def rl(x, d):
        return (x << np.uint32(d)) | (x >> np.uint32(32 - d))

    def rounds(x0, x1, rots):
        for r in rots:
            x0 = x0 + x1
            x1 = rl(x1, r)
            x1 = x1 ^ x0
        return x0, x1

    ks0 = np.uint32(k1)
    ks1 = np.uint32(k2)
    ks2 = np.uint32(0x1BD11BDA) ^ ks0 ^ ks1
    x0 = x0 + ks0
    x1 = x1 + ks1
    x0, x1 = rounds(x0, x1, r1)
    x0 = x0 + ks1
    x1 = x1 + (ks2 + np.uint32(1))
    x0, x1 = rounds(x0, x1, r2)
    x0 = x0 + ks2
    x1 = x1 + (ks0 + np.uint32(2))
    x0, x1 = rounds(x0, x1, r1)
    x0 = x0 + ks0
    x1 = x1 + (ks1 + np.uint32(3))
    x0, x1 = rounds(x0, x1, r2)
    x0 = x0 + ks1
    x1 = x1 + (ks2 + np.uint32(4))
    x0, x1 = rounds(x0, x1, r1)
    x0 = x0 + ks2
    x1 = x1 + (ks0 + np.uint32(5))
    return x0, x1


def _noise_recip():
    if not _consts:
        o0, o1 = _threefry2x32_np(np.uint32(0), np.uint32(0),
                                  np.uint32([0]), np.uint32([12345]))
        k1, k2 = o0[0], o1[0]
        idx = np.arange(_B * _V, dtype=np.uint32)
        b0, b1 = _threefry2x32_np(k1, k2, np.zeros_like(idx), idx)
        bits = b0 ^ b1
        fb = (bits >> np.uint32(9)) | np.uint32(0x3F800000)
        u = fb.view(np.float32) - np.float32(1.0)
        n = -np.log1p(-u)
        g = (-np.log(n.astype(np.float64) + _EPS)).astype(np.float32)
        _consts.append(g)
    return _consts[0]


def _sampler_sc_body(logits_hbm, gumb_hbm, temps_hbm, out_hbm,
                     lbuf0, gbuf0, lbuf1, gbuf1, ltail, gtail, tbuf, obuf,
                     sl0, sg0, sl1, sg1, slt, sgt):
    wid = lax.axis_index("s") * _NC + lax.axis_index("c")
    grp = pl.multiple_of(8 * (wid // 2), 8)
    half = wid % 2
    pltpu.sync_copy(temps_hbm, tbuf)
    lane = lax.iota(jnp.int32, 16)
    neg_inf = jnp.float32(-jnp.inf)

    lbufs = [(lbuf0, sl0), (lbuf1, sl1)]
    gbufs = [(gbuf0, sg0), (gbuf1, sg1)]

    def issue(c, par):
        col = pl.multiple_of(c * _CH, 128)
        lb, sl = lbufs[par]
        gb, sg = gbufs[par]
        pltpu.async_copy(logits_hbm.at[pl.ds(grp, 8), pl.ds(col, _CH)],
                         lb, sl)
        for j in range(_RPW):
            off = pl.multiple_of((wid * _RPW + j) * _V + c * _CH, 8)
            pltpu.async_copy(gumb_hbm.at[pl.ds(off, _CH)],
                             gb.at[pl.ds(j * _CH, _CH)], sg)

    def wait_chunk(c, par):
        col = pl.multiple_of(c * _CH, 128)
        lb, sl = lbufs[par]
        gb, sg = gbufs[par]
        pltpu.make_async_copy(
            logits_hbm.at[pl.ds(grp, 8), pl.ds(col, _CH)], lb, sl).wait()
        for j in range(_RPW):
            pltpu.make_async_copy(
                gumb_hbm.at[pl.ds(0, _CH)], gb.at[pl.ds(j * _CH, _CH)],
                sg).wait()

    def sweep(lb, gb, jrow, j, tv, m_vec, base, nv, carry, gstride):
        def body(i, carry1):
            mv, sb, si = carry1
            sl_ = pl.ds(i * 16, 16)
            l = lb[jrow, sl_]
            a = l + tv * gb[pl.ds(j * gstride + i * 16, 16)]
            s = jnp.exp(a - m_vec)
            vidx = (base + i * 16) + lane
            upd = s > sb
            return (jnp.maximum(mv, a),
                    jnp.where(upd, s, sb),
                    jnp.where(upd, vidx, si))
        return lax.fori_loop(0, nv, body, carry, unroll=5)

    def proc(c, par, st, tail=False):
        if tail:
            lb, gb = ltail, gtail
            base = _NFULL * _CH
            nv = _NVT
        else:
            lb, _ = lbufs[par]
            gb, _ = gbufs[par]
            base = c * _CH
            nv = _NV
        new_st = []
        for j in range(_RPW):
            m_norm, sbest, sidx = st[j]
            row = wid * _RPW + j
            tv = plsc.load_gather(tbuf, [jnp.full((16,), row, jnp.int32)])
            jrow = 4 * half + j
            mv0 = jnp.full((16,), neg_inf, jnp.float32)
            mv, sb1, si1 = sweep(
                lb, gb, jrow, j, tv, jnp.full((16,), m_norm, jnp.float32),
                base, nv, (mv0, sbest, sidx), nv * 16)
            m_new = jnp.max(mv)

            def no_redo(_, sb1=sb1, si1=si1, m_norm=m_norm):
                return m_norm, sb1, si1

            def redo(_, lb=lb, gb=gb, jrow=jrow, j=j, tv=tv, base=base,
                     nv=nv, mv=mv, m_new=m_new, m_norm=m_norm, sbest=sbest,
                     sidx=sidx):
                sb0 = sbest * jnp.exp(
                    jnp.full((16,), m_norm - m_new, jnp.float32))
                _, sb2, si2 = sweep(
                    lb, gb, jrow, j, tv,
                    jnp.full((16,), m_new, jnp.float32),
                    base, nv, (mv, sb0, sidx), nv * 16)
                return m_new, sb2, si2

            m2, sb2, si2 = lax.cond(m_new > m_norm + jnp.float32(80.0),
                                    redo, no_redo, 0)
            new_st.append((m2, sb2, si2))
        return tuple(new_st)

    st = tuple((neg_inf,
                jnp.zeros((16,), jnp.float32), jnp.zeros((16,), jnp.int32))
               for _ in range(_RPW))

    issue(0, 0)
    issue(1, 1)

    def loop_body(k, st):
        c0 = 2 * k
        wait_chunk(c0, 0)
        st = proc(c0, 0, st)
        issue(c0 + 2, 0)
        wait_chunk(c0 + 1, 1)
        st = proc(c0 + 1, 1, st)
        issue(c0 + 3, 1)
        return st

    st = lax.fori_loop(0, 15, loop_body, st)

    wait_chunk(30, 0)
    st = proc(30, 0, st)
    issue(32, 0)
    wait_chunk(31, 1)
    st = proc(31, 1, st)
    tcol = _NFULL * _CH
    pltpu.async_copy(logits_hbm.at[pl.ds(grp, 8), pl.ds(tcol, _TAIL)],
                     ltail, slt)
    for j in range(_RPW):
        toff = pl.multiple_of((wid * _RPW + j) * _V + tcol, 8)
        pltpu.async_copy(gumb_hbm.at[pl.ds(toff, _TAIL)],
                         gtail.at[pl.ds(j * _TAIL, _TAIL)], sgt)
    wait_chunk(32, 0)
    st = proc(32, 0, st)
    pltpu.make_async_copy(logits_hbm.at[pl.ds(grp, 8), pl.ds(tcol, _TAIL)],
                          ltail, slt).wait()
    for j in range(_RPW):
        pltpu.make_async_copy(gumb_hbm.at[pl.ds(0, _TAIL)],
                              gtail.at[pl.ds(j * _TAIL, _TAIL)], sgt).wait()
    st = proc(0, 0, st, tail=True)

    tokens = jnp.zeros((16,), jnp.int32)
    ibig = jnp.full((16,), _IBIG, jnp.int32)
    for j in range(_RPW):
        _, sbest, sidx = st[j]
        smax = jnp.full((16,), jnp.max(sbest), jnp.float32)
        stok = jnp.min(jnp.where(sbest == smax, sidx, ibig))
        tokens = jnp.where(lane == j, jnp.full((16,), stok, jnp.int32),
                           tokens)

    obuf[...] = tokens
    pltpu.sync_copy(obuf, out_hbm.at[pl.ds(wid * _OUTP, _OUTP)])


_sampler_cache = []


def _sampler_sc():
    if not _sampler_cache:
        _sampler_cache.append(pl.kernel(
            _sampler_sc_body,
            out_type=jax.ShapeDtypeStruct((_NW * _OUTP,), jnp.int32),
            mesh=plsc.VectorSubcoreMesh(core_axis_name="c",
                                        subcore_axis_name="s",
                                        num_cores=_NC, num_subcores=_NS),
            scratch_types=[
                pltpu.VMEM((8, _CH), jnp.float32),
                pltpu.VMEM((_RPW * _CH,), jnp.float32),
                pltpu.VMEM((8, _CH), jnp.float32),
                pltpu.VMEM((_RPW * _CH,), jnp.float32),
                pltpu.VMEM((8, _TAIL), jnp.float32),
                pltpu.VMEM((_RPW * _TAIL,), jnp.float32),
                pltpu.VMEM((_B,), jnp.float32),
                pltpu.VMEM((_OUTP,), jnp.int32),
                pltpu.SemaphoreType.DMA,
                pltpu.SemaphoreType.DMA,
                pltpu.SemaphoreType.DMA,
                pltpu.SemaphoreType.DMA,
                pltpu.SemaphoreType.DMA,
                pltpu.SemaphoreType.DMA,
            ],
            compiler_params=pltpu.CompilerParams(needs_layout_passes=False),
        ))
    return _sampler_cache[0]


def kernel(logits, temperatures):
    gumb = jnp.asarray(_noise_recip())
    flat = _sampler_sc()(logits, gumb, temperatures)
    return flat.reshape(_NW, _OUTP)[:, :_RPW].reshape(_B)
